# Initial kernel scaffold; baseline (speedup 1.0000x reference)
import jax
import jax.numpy as jnp
from jax.experimental import pallas as pl


def kernel(encoder_embedding, embedding_weight):
    raise NotImplementedError("write your pallas kernel here")


# speedup vs baseline: 1.8796x; 1.8796x over previous
import functools

import jax
import jax.numpy as jnp
from jax.experimental import pallas as pl

K = 1024
D = 10
N = 8192
BLK = 1024
GRID = N // BLK


def _vq_kernel(xt_ref, x_ref, w_ref, out_ref, loss_ref):
    pid = pl.program_id(0)
    w = w_ref[...]
    xt = xt_ref[...]
    acc = jnp.zeros((K, BLK), dtype=jnp.float32)
    for d in range(D):
        diff = w[:, d][:, None] - xt[d, :][None, :]
        acc = acc + diff * diff
    idx = jnp.argmin(acc, axis=0)
    onehot = (jax.lax.broadcasted_iota(jnp.int32, (K, BLK), 0)
              == idx[None, :]).astype(jnp.float32)
    q = jax.lax.dot_general(
        onehot, w,
        dimension_numbers=(((0,), (0,)), ((), ())),
        preferred_element_type=jnp.float32)
    x = x_ref[...]
    out_ref[...] = x + (q - x)
    partial = jnp.sum((x - q) ** 2).reshape(1, 1)

    @pl.when(pid == 0)
    def _():
        loss_ref[...] = jnp.zeros((1, 1), jnp.float32)

    loss_ref[...] += partial

    @pl.when(pid == GRID - 1)
    def _():
        loss_ref[...] = loss_ref[...] / (N * D)


@jax.jit
def kernel(encoder_embedding, embedding_weight):
    xt = encoder_embedding.T
    out, loss = pl.pallas_call(
        _vq_kernel,
        grid=(GRID,),
        in_specs=[
            pl.BlockSpec((D, BLK), lambda i: (0, i)),
            pl.BlockSpec((BLK, D), lambda i: (i, 0)),
            pl.BlockSpec((K, D), lambda i: (0, 0)),
        ],
        out_specs=[
            pl.BlockSpec((BLK, D), lambda i: (i, 0)),
            pl.BlockSpec((1, 1), lambda i: (0, 0)),
        ],
        out_shape=[
            jax.ShapeDtypeStruct((N, D), jnp.float32),
            jax.ShapeDtypeStruct((1, 1), jnp.float32),
        ],
    )(xt, encoder_embedding, embedding_weight)
    return out, loss[0, 0]
